# baseline (speedup 1.0000x reference)
import jax
import jax.numpy as jnp
from jax import lax
from jax.experimental import pallas as pl
from jax.experimental.pallas import tpu as pltpu
from jax.experimental.pallas import tpu_sc as plsc

_B = 4096
_L = 200
_D = 64
_DP = 128
_TOT = _B * _L
_NW = 32
_PER_W = _TOT // _NW
_IROWS = _PER_W // 128
_CHUNK = 256
_NCH = _PER_W // _CHUNK
_KSUB = _CHUNK // 128


def _gathers(tab_hbm, idx_v, rows, gsem, c):
    return [
        pltpu.make_async_copy(
            tab_hbm.at[idx_v.at[_KSUB * c + j]],
            rows.at[pl.ds(j * 128, 128)],
            gsem,
        )
        for j in range(_KSUB)
    ]


def _write(out_hbm, rows, wsem, base, c):
    return pltpu.make_async_copy(
        rows,
        out_hbm.at[pl.ds(base + c * _CHUNK, _CHUNK)],
        wsem,
    )


def _mask(idx_v, rows, c, lanes, zeros16):
    for r in range(_KSUB):
        def mask_group(g, carry, r=r):
            idxv = idx_v[_KSUB * c + r, pl.ds(g * 16, 16)]
            m = idxv == 0
            rowi = (r * 128 + g * 16) + lanes
            coli = jnp.zeros((16,), jnp.int32)
            for _ in range(_D):
                plsc.store_scatter(rows, [rowi, coli], zeros16, mask=m)
                coli = coli + 1
            return carry

        lax.fori_loop(0, 8, mask_group, None)


def _body(idx_hbm, tab_hbm, out_hbm, idx_v, rows0, rows1, gsem0, gsem1,
          wsem0, wsem1):
    wid = lax.axis_index("s") * 2 + lax.axis_index("c")
    base = wid * _PER_W

    lanes = lax.iota(jnp.int32, 16)
    zeros16 = jnp.zeros((16,), jnp.float32)

    pltpu.sync_copy(idx_hbm.at[wid], idx_v)
    for cp in _gathers(tab_hbm, idx_v, rows0, gsem0, 0):
        cp.start()

    def step(g, carry):
        c0 = 2 * g
        for cp in _gathers(tab_hbm, idx_v, rows0, gsem0, c0):
            cp.wait()

        @pl.when(g >= 1)
        def _():
            _write(out_hbm, rows1, wsem1, base, c0 - 1).wait()

        for cp in _gathers(tab_hbm, idx_v, rows1, gsem1, c0 + 1):
            cp.start()
        _mask(idx_v, rows0, c0, lanes, zeros16)
        _write(out_hbm, rows0, wsem0, base, c0).start()

        c1 = 2 * g + 1
        for cp in _gathers(tab_hbm, idx_v, rows1, gsem1, c1):
            cp.wait()
        _write(out_hbm, rows0, wsem0, base, c1 - 1).wait()

        @pl.when(g < _NCH // 2 - 1)
        def _():
            for cp in _gathers(tab_hbm, idx_v, rows0, gsem0, c1 + 1):
                cp.start()

        _mask(idx_v, rows1, c1, lanes, zeros16)
        _write(out_hbm, rows1, wsem1, base, c1).start()
        return carry

    lax.fori_loop(0, _NCH // 2, step, None)
    _write(out_hbm, rows1, wsem1, base, _NCH - 1).wait()


_V = 1000000
_TC_C = 1024


def _tp_body(int_ref, out_ref):
    out_ref[:, 0:_D] = int_ref[...].T


def _transpose_pad(embT):
    return pl.pallas_call(
        _tp_body,
        grid=((_V + _TC_C - 1) // _TC_C,),
        in_specs=[pl.BlockSpec((_D, _TC_C), lambda i: (0, i))],
        out_specs=pl.BlockSpec((_TC_C, _DP), lambda i: (i, 0)),
        out_shape=jax.ShapeDtypeStruct((_V, _DP), jnp.float32),
    )(embT)


def kernel(inputs, embeddings):
    idx = inputs.reshape(_TOT).astype(jnp.int32)
    idx = idx.reshape(_NW, _IROWS, 128)
    tab = _transpose_pad(embeddings.T)
    mesh = plsc.VectorSubcoreMesh(core_axis_name="c", subcore_axis_name="s")
    out = pl.kernel(
        _body,
        mesh=mesh,
        compiler_params=pltpu.CompilerParams(
            needs_layout_passes=False,
            use_tc_tiling_on_sc=True,
        ),
        out_type=jax.ShapeDtypeStruct((_TOT, _DP), jnp.float32),
        scratch_types=[
            pltpu.VMEM((_IROWS, 128), jnp.int32),
            pltpu.VMEM((_CHUNK, _DP), jnp.float32),
            pltpu.VMEM((_CHUNK, _DP), jnp.float32),
            pltpu.SemaphoreType.DMA,
            pltpu.SemaphoreType.DMA,
            pltpu.SemaphoreType.DMA,
            pltpu.SemaphoreType.DMA,
        ],
    )(idx, tab)
    return out[:, :_D].reshape(_B, _L, _D)


# speedup vs baseline: 1.4503x; 1.4503x over previous
import jax
import jax.numpy as jnp
from jax import lax
from jax.experimental import pallas as pl
from jax.experimental.pallas import tpu as pltpu
from jax.experimental.pallas import tpu_sc as plsc

_B = 4096
_L = 200
_D = 64
_DP = 128
_TOT = _B * _L
_NW = 32
_PER_W = _TOT // _NW
_IROWS = _PER_W // 128
_CHUNK = 256
_NCH = _PER_W // _CHUNK
_KSUB = _CHUNK // 128


def _gathers(tab_hbm, idx_v, rows, gsem, c):
    return [
        pltpu.make_async_copy(
            tab_hbm.at[idx_v.at[_KSUB * c + j]],
            rows.at[pl.ds(j * 128, 128)],
            gsem,
        )
        for j in range(_KSUB)
    ]


def _write(out_hbm, rows, wsem, base, c):
    return pltpu.make_async_copy(
        rows,
        out_hbm.at[pl.ds(base + c * _CHUNK, _CHUNK)],
        wsem,
    )


def _mask(idx_v, rows, c, lanes, zeros16):
    for r in range(_KSUB):
        def mask_group(g, carry, r=r):
            idxv = idx_v[_KSUB * c + r, pl.ds(g * 16, 16)]
            m = idxv == 0
            rowi = (r * 128 + g * 16) + lanes
            coli = jnp.zeros((16,), jnp.int32)
            for _ in range(_D):
                plsc.store_scatter(rows, [rowi, coli], zeros16, mask=m)
                coli = coli + 1
            return carry

        lax.fori_loop(0, 8, mask_group, None)


def _body(idx_hbm, tab_hbm, out_hbm, idx_v, rows0, rows1, gsem0, gsem1,
          wsem0, wsem1):
    wid = lax.axis_index("s") * 2 + lax.axis_index("c")
    base = wid * _PER_W

    lanes = lax.iota(jnp.int32, 16)
    zeros16 = jnp.zeros((16,), jnp.float32)

    pltpu.sync_copy(idx_hbm.at[wid], idx_v)
    for cp in _gathers(tab_hbm, idx_v, rows0, gsem0, 0):
        cp.start()

    def step(g, carry):
        c0 = 2 * g
        for cp in _gathers(tab_hbm, idx_v, rows0, gsem0, c0):
            cp.wait()

        @pl.when(g >= 1)
        def _():
            _write(out_hbm, rows1, wsem1, base, c0 - 1).wait()

        for cp in _gathers(tab_hbm, idx_v, rows1, gsem1, c0 + 1):
            cp.start()
        _mask(idx_v, rows0, c0, lanes, zeros16)
        _write(out_hbm, rows0, wsem0, base, c0).start()

        c1 = 2 * g + 1
        for cp in _gathers(tab_hbm, idx_v, rows1, gsem1, c1):
            cp.wait()
        _write(out_hbm, rows0, wsem0, base, c1 - 1).wait()

        @pl.when(g < _NCH // 2 - 1)
        def _():
            for cp in _gathers(tab_hbm, idx_v, rows0, gsem0, c1 + 1):
                cp.start()

        _mask(idx_v, rows1, c1, lanes, zeros16)
        _write(out_hbm, rows1, wsem1, base, c1).start()
        return carry

    lax.fori_loop(0, _NCH // 2, step, None)
    _write(out_hbm, rows1, wsem1, base, _NCH - 1).wait()


_V = 1000000
_TC_C = 4096


def _tp_body(int_ref, out_ref):
    out_ref[:, 0:_D] = int_ref[...].T


def _transpose_pad(embT):
    return pl.pallas_call(
        _tp_body,
        grid=((_V + _TC_C - 1) // _TC_C,),
        in_specs=[pl.BlockSpec((_D, _TC_C), lambda i: (0, i))],
        out_specs=pl.BlockSpec((_TC_C, _DP), lambda i: (i, 0)),
        out_shape=jax.ShapeDtypeStruct((_V, _DP), jnp.float32),
    )(embT)


def kernel(inputs, embeddings):
    idx = inputs.reshape(_TOT).astype(jnp.int32)
    idx = idx.reshape(_NW, _IROWS, 128)
    tab = _transpose_pad(embeddings.T)
    mesh = plsc.VectorSubcoreMesh(core_axis_name="c", subcore_axis_name="s")
    out = pl.kernel(
        _body,
        mesh=mesh,
        compiler_params=pltpu.CompilerParams(
            needs_layout_passes=False,
            use_tc_tiling_on_sc=True,
        ),
        out_type=jax.ShapeDtypeStruct((_TOT, _DP), jnp.float32),
        scratch_types=[
            pltpu.VMEM((_IROWS, 128), jnp.int32),
            pltpu.VMEM((_CHUNK, _DP), jnp.float32),
            pltpu.VMEM((_CHUNK, _DP), jnp.float32),
            pltpu.SemaphoreType.DMA,
            pltpu.SemaphoreType.DMA,
            pltpu.SemaphoreType.DMA,
            pltpu.SemaphoreType.DMA,
        ],
    )(idx, tab)
    return out[:, :_D].reshape(_B, _L, _D)


# speedup vs baseline: 1.5907x; 1.0968x over previous
import jax
import jax.numpy as jnp
from jax import lax
from jax.experimental import pallas as pl
from jax.experimental.pallas import tpu as pltpu
from jax.experimental.pallas import tpu_sc as plsc

_B = 4096
_L = 200
_D = 64
_DP = 128
_TOT = _B * _L
_NW = 32
_PER_W = _TOT // _NW
_IROWS = _PER_W // 128
_CHUNK = 256
_NCH = _PER_W // _CHUNK
_KSUB = _CHUNK // 128


def _gathers(tab_hbm, idx_v, rows, gsem, c):
    return [
        pltpu.make_async_copy(
            tab_hbm.at[idx_v.at[_KSUB * c + j]],
            rows.at[pl.ds(j * 128, 128)],
            gsem,
        )
        for j in range(_KSUB)
    ]


def _write(out_hbm, rows, wsem, base, c):
    return pltpu.make_async_copy(
        rows,
        out_hbm.at[pl.ds(base + c * _CHUNK, _CHUNK)],
        wsem,
    )


def _mask(idx_v, rows, c, lanes, zeros16):
    for r in range(_KSUB):
        def mask_group(g, carry, r=r):
            idxv = idx_v[_KSUB * c + r, pl.ds(g * 16, 16)]
            m = idxv == 0
            rowi = (r * 128 + g * 16) + lanes
            coli = jnp.zeros((16,), jnp.int32)
            for _ in range(_D):
                plsc.store_scatter(rows, [rowi, coli], zeros16, mask=m)
                coli = coli + 1
            return carry

        lax.fori_loop(0, 8, mask_group, None)


def _body(idx_hbm, tab_hbm, out_hbm, idx_v, rows0, rows1, gsem0, gsem1,
          wsem0, wsem1):
    wid = lax.axis_index("s") * 2 + lax.axis_index("c")
    base = wid * _PER_W

    lanes = lax.iota(jnp.int32, 16)
    zeros16 = jnp.zeros((16,), jnp.float32)

    pltpu.sync_copy(idx_hbm.at[wid], idx_v)
    for cp in _gathers(tab_hbm, idx_v, rows0, gsem0, 0):
        cp.start()

    def step(g, carry):
        c0 = 2 * g
        for cp in _gathers(tab_hbm, idx_v, rows0, gsem0, c0):
            cp.wait()

        @pl.when(g >= 1)
        def _():
            _write(out_hbm, rows1, wsem1, base, c0 - 1).wait()

        for cp in _gathers(tab_hbm, idx_v, rows1, gsem1, c0 + 1):
            cp.start()
        _mask(idx_v, rows0, c0, lanes, zeros16)
        _write(out_hbm, rows0, wsem0, base, c0).start()

        c1 = 2 * g + 1
        for cp in _gathers(tab_hbm, idx_v, rows1, gsem1, c1):
            cp.wait()
        _write(out_hbm, rows0, wsem0, base, c1 - 1).wait()

        @pl.when(g < _NCH // 2 - 1)
        def _():
            for cp in _gathers(tab_hbm, idx_v, rows0, gsem0, c1 + 1):
                cp.start()

        _mask(idx_v, rows1, c1, lanes, zeros16)
        _write(out_hbm, rows1, wsem1, base, c1).start()
        return carry

    lax.fori_loop(0, _NCH // 2, step, None)
    _write(out_hbm, rows1, wsem1, base, _NCH - 1).wait()


_V = 1000000
_TC_C = 8192


def _tp_body(int_ref, out_ref):
    out_ref[:, 0:_D] = int_ref[...].T


def _transpose_pad(embT):
    return pl.pallas_call(
        _tp_body,
        grid=((_V + _TC_C - 1) // _TC_C,),
        in_specs=[pl.BlockSpec((_D, _TC_C), lambda i: (0, i))],
        out_specs=pl.BlockSpec((_TC_C, _DP), lambda i: (i, 0)),
        out_shape=jax.ShapeDtypeStruct((_V, _DP), jnp.float32),
    )(embT)


def kernel(inputs, embeddings):
    idx = inputs.reshape(_TOT).astype(jnp.int32)
    idx = idx.reshape(_NW, _IROWS, 128)
    tab = _transpose_pad(embeddings.T)
    mesh = plsc.VectorSubcoreMesh(core_axis_name="c", subcore_axis_name="s")
    out = pl.kernel(
        _body,
        mesh=mesh,
        compiler_params=pltpu.CompilerParams(
            needs_layout_passes=False,
            use_tc_tiling_on_sc=True,
        ),
        out_type=jax.ShapeDtypeStruct((_TOT, _DP), jnp.float32),
        scratch_types=[
            pltpu.VMEM((_IROWS, 128), jnp.int32),
            pltpu.VMEM((_CHUNK, _DP), jnp.float32),
            pltpu.VMEM((_CHUNK, _DP), jnp.float32),
            pltpu.SemaphoreType.DMA,
            pltpu.SemaphoreType.DMA,
            pltpu.SemaphoreType.DMA,
            pltpu.SemaphoreType.DMA,
        ],
    )(idx, tab)
    return out[:, :_D].reshape(_B, _L, _D)


# speedup vs baseline: 1.6342x; 1.0274x over previous
import jax
import jax.numpy as jnp
from jax import lax
from jax.experimental import pallas as pl
from jax.experimental.pallas import tpu as pltpu
from jax.experimental.pallas import tpu_sc as plsc

_B = 4096
_L = 200
_D = 64
_DP = 128
_TOT = _B * _L
_NW = 32
_PER_W = _TOT // _NW
_IROWS = _PER_W // 128
_CHUNK = 256
_NCH = _PER_W // _CHUNK
_KSUB = _CHUNK // 128


def _gathers(tab_hbm, idx_v, rows, gsem, c):
    return [
        pltpu.make_async_copy(
            tab_hbm.at[idx_v.at[_KSUB * c + j]],
            rows.at[pl.ds(j * 128, 128)],
            gsem,
        )
        for j in range(_KSUB)
    ]


def _write(out_hbm, rows, wsem, base, c):
    return pltpu.make_async_copy(
        rows,
        out_hbm.at[pl.ds(base + c * _CHUNK, _CHUNK)],
        wsem,
    )


def _mask(idx_v, rows, c, lanes, zeros16):
    for r in range(_KSUB):
        def mask_group(g, carry, r=r):
            idxv = idx_v[_KSUB * c + r, pl.ds(g * 16, 16)]
            m = idxv == 0
            rowi = (r * 128 + g * 16) + lanes
            coli = jnp.zeros((16,), jnp.int32)
            for _ in range(_D):
                plsc.store_scatter(rows, [rowi, coli], zeros16, mask=m)
                coli = coli + 1
            return carry

        lax.fori_loop(0, 8, mask_group, None)


def _body(idx_hbm, tab_hbm, out_hbm, idx_v, rows0, rows1, gsem0, gsem1,
          wsem0, wsem1):
    wid = lax.axis_index("s") * 2 + lax.axis_index("c")
    base = wid * _PER_W

    lanes = lax.iota(jnp.int32, 16)
    zeros16 = jnp.zeros((16,), jnp.float32)

    pltpu.sync_copy(idx_hbm.at[wid], idx_v)
    for cp in _gathers(tab_hbm, idx_v, rows0, gsem0, 0):
        cp.start()

    def step(g, carry):
        c0 = 2 * g
        for cp in _gathers(tab_hbm, idx_v, rows0, gsem0, c0):
            cp.wait()

        @pl.when(g >= 1)
        def _():
            _write(out_hbm, rows1, wsem1, base, c0 - 1).wait()

        for cp in _gathers(tab_hbm, idx_v, rows1, gsem1, c0 + 1):
            cp.start()
        _mask(idx_v, rows0, c0, lanes, zeros16)
        _write(out_hbm, rows0, wsem0, base, c0).start()

        c1 = 2 * g + 1
        for cp in _gathers(tab_hbm, idx_v, rows1, gsem1, c1):
            cp.wait()
        _write(out_hbm, rows0, wsem0, base, c1 - 1).wait()

        @pl.when(g < _NCH // 2 - 1)
        def _():
            for cp in _gathers(tab_hbm, idx_v, rows0, gsem0, c1 + 1):
                cp.start()

        _mask(idx_v, rows1, c1, lanes, zeros16)
        _write(out_hbm, rows1, wsem1, base, c1).start()
        return carry

    lax.fori_loop(0, _NCH // 2, step, None)
    _write(out_hbm, rows1, wsem1, base, _NCH - 1).wait()


_V = 1000000
_TC_C = 16384


def _tp_body(int_ref, out_ref):
    out_ref[:, 0:_D] = int_ref[...].T


def _transpose_pad(embT):
    return pl.pallas_call(
        _tp_body,
        grid=((_V + _TC_C - 1) // _TC_C,),
        in_specs=[pl.BlockSpec((_D, _TC_C), lambda i: (0, i))],
        out_specs=pl.BlockSpec((_TC_C, _DP), lambda i: (i, 0)),
        out_shape=jax.ShapeDtypeStruct((_V, _DP), jnp.float32),
    )(embT)


def kernel(inputs, embeddings):
    idx = inputs.reshape(_TOT).astype(jnp.int32)
    idx = idx.reshape(_NW, _IROWS, 128)
    tab = _transpose_pad(embeddings.T)
    mesh = plsc.VectorSubcoreMesh(core_axis_name="c", subcore_axis_name="s")
    out = pl.kernel(
        _body,
        mesh=mesh,
        compiler_params=pltpu.CompilerParams(
            needs_layout_passes=False,
            use_tc_tiling_on_sc=True,
        ),
        out_type=jax.ShapeDtypeStruct((_TOT, _DP), jnp.float32),
        scratch_types=[
            pltpu.VMEM((_IROWS, 128), jnp.int32),
            pltpu.VMEM((_CHUNK, _DP), jnp.float32),
            pltpu.VMEM((_CHUNK, _DP), jnp.float32),
            pltpu.SemaphoreType.DMA,
            pltpu.SemaphoreType.DMA,
            pltpu.SemaphoreType.DMA,
            pltpu.SemaphoreType.DMA,
        ],
    )(idx, tab)
    return out[:, :_D].reshape(_B, _L, _D)


# speedup vs baseline: 1.6493x; 1.0093x over previous
import jax
import jax.numpy as jnp
from jax import lax
from jax.experimental import pallas as pl
from jax.experimental.pallas import tpu as pltpu
from jax.experimental.pallas import tpu_sc as plsc

_B = 4096
_L = 200
_D = 64
_DP = 128
_TOT = _B * _L
_NW = 32
_PER_W = _TOT // _NW
_IROWS = _PER_W // 128
_CHUNK = 256
_NCH = _PER_W // _CHUNK
_KSUB = _CHUNK // 128


def _gathers(tab_hbm, idx_v, rows, gsem, c):
    return [
        pltpu.make_async_copy(
            tab_hbm.at[idx_v.at[_KSUB * c + j]],
            rows.at[pl.ds(j * 128, 128)],
            gsem,
        )
        for j in range(_KSUB)
    ]


def _write(out_hbm, rows, wsem, base, c):
    return pltpu.make_async_copy(
        rows,
        out_hbm.at[pl.ds(base + c * _CHUNK, _CHUNK)],
        wsem,
    )


def _mask(idx_v, rows, c, lanes, zeros16):
    for r in range(_KSUB):
        def mask_group(g, carry, r=r):
            idxv = idx_v[_KSUB * c + r, pl.ds(g * 16, 16)]
            m = idxv == 0
            rowi = (r * 128 + g * 16) + lanes
            coli = jnp.zeros((16,), jnp.int32)
            for _ in range(_D):
                plsc.store_scatter(rows, [rowi, coli], zeros16, mask=m)
                coli = coli + 1
            return carry

        lax.fori_loop(0, 8, mask_group, None)


def _body(idx_hbm, tab_hbm, out_hbm, idx_v, rows0, rows1, gsem0, gsem1,
          wsem0, wsem1):
    wid = lax.axis_index("s") * 2 + lax.axis_index("c")
    base = wid * _PER_W

    lanes = lax.iota(jnp.int32, 16)
    zeros16 = jnp.zeros((16,), jnp.float32)

    pltpu.sync_copy(idx_hbm.at[wid], idx_v)
    for cp in _gathers(tab_hbm, idx_v, rows0, gsem0, 0):
        cp.start()

    def step(g, carry):
        c0 = 2 * g
        for cp in _gathers(tab_hbm, idx_v, rows0, gsem0, c0):
            cp.wait()

        @pl.when(g >= 1)
        def _():
            _write(out_hbm, rows1, wsem1, base, c0 - 1).wait()

        for cp in _gathers(tab_hbm, idx_v, rows1, gsem1, c0 + 1):
            cp.start()
        _mask(idx_v, rows0, c0, lanes, zeros16)
        _write(out_hbm, rows0, wsem0, base, c0).start()

        c1 = 2 * g + 1
        for cp in _gathers(tab_hbm, idx_v, rows1, gsem1, c1):
            cp.wait()
        _write(out_hbm, rows0, wsem0, base, c1 - 1).wait()

        @pl.when(g < _NCH // 2 - 1)
        def _():
            for cp in _gathers(tab_hbm, idx_v, rows0, gsem0, c1 + 1):
                cp.start()

        _mask(idx_v, rows1, c1, lanes, zeros16)
        _write(out_hbm, rows1, wsem1, base, c1).start()
        return carry

    lax.fori_loop(0, _NCH // 2, step, None)
    _write(out_hbm, rows1, wsem1, base, _NCH - 1).wait()


_V = 1000000
_TC_C = 32768


def _tp_body(int_ref, out_ref):
    out_ref[:, 0:_D] = int_ref[...].T


def _transpose_pad(embT):
    return pl.pallas_call(
        _tp_body,
        grid=((_V + _TC_C - 1) // _TC_C,),
        in_specs=[pl.BlockSpec((_D, _TC_C), lambda i: (0, i))],
        out_specs=pl.BlockSpec((_TC_C, _DP), lambda i: (i, 0)),
        out_shape=jax.ShapeDtypeStruct((_V, _DP), jnp.float32),
    )(embT)


def kernel(inputs, embeddings):
    idx = inputs.reshape(_TOT).astype(jnp.int32)
    idx = idx.reshape(_NW, _IROWS, 128)
    tab = _transpose_pad(embeddings.T)
    mesh = plsc.VectorSubcoreMesh(core_axis_name="c", subcore_axis_name="s")
    out = pl.kernel(
        _body,
        mesh=mesh,
        compiler_params=pltpu.CompilerParams(
            needs_layout_passes=False,
            use_tc_tiling_on_sc=True,
        ),
        out_type=jax.ShapeDtypeStruct((_TOT, _DP), jnp.float32),
        scratch_types=[
            pltpu.VMEM((_IROWS, 128), jnp.int32),
            pltpu.VMEM((_CHUNK, _DP), jnp.float32),
            pltpu.VMEM((_CHUNK, _DP), jnp.float32),
            pltpu.SemaphoreType.DMA,
            pltpu.SemaphoreType.DMA,
            pltpu.SemaphoreType.DMA,
            pltpu.SemaphoreType.DMA,
        ],
    )(idx, tab)
    return out[:, :_D].reshape(_B, _L, _D)
